# Initial kernel scaffold; baseline (speedup 1.0000x reference)
import functools

import jax
import jax.numpy as jnp
from jax import lax
from jax.experimental import pallas as pl
from jax.experimental.pallas import tpu as pltpu
from jax.experimental.pallas import tpu_sc as plsc

_N = 10000
_E = 320000
_D = 128
_NC = 2
_NS = 16
_NW = _NC * _NS
_EPW = _E // _NW
_C = 80
_NCHUNK = _EPW // _C
_RPT = _N // _NS
_TC_BR = 1250


def _tanh_body(x_ref, o_ref):
    o_ref[...] = jnp.tanh(x_ref[...])


def _add_body(a_ref, b_ref, o_ref):
    o_ref[...] = a_ref[0] + b_ref[0]


_sc_mesh = plsc.VectorSubcoreMesh(core_axis_name="c", subcore_axis_name="s")


@functools.partial(
    pl.kernel,
    out_type=jax.ShapeDtypeStruct((_NC, _N, _D), jnp.float32),
    mesh=_sc_mesh,
    scratch_types=[
        pltpu.VMEM((_C,), jnp.int32),
        pltpu.VMEM((_C,), jnp.int32),
        pltpu.VMEM((_C,), jnp.int32),
        pltpu.VMEM((_C,), jnp.float32),
        pltpu.VMEM((_C, _D), jnp.float32),
        pltpu.VMEM_SHARED((_N, _D), jnp.float32),
        pltpu.SemaphoreType.DMA,
        pltpu.SemaphoreType.DMA,
    ],
)
def _sc_scatter(t_hbm, src_hbm, dst_hbm, wflat_hbm, out_hbm,
                src_v, dst_v, widx_v, wv_v, rows_v, acc_sh, sem0, sem1):
    c = lax.axis_index("c")
    s = lax.axis_index("s")
    wid = c * _NS + s

    def _zrow(e, carry):
        z = jnp.zeros((16,), jnp.float32)
        for j in range(_D // 16):
            rows_v[e, pl.ds(j * 16, 16)] = z
        return carry

    lax.fori_loop(0, _C, _zrow, 0)
    zbase = s * _RPT
    for r in range(_RPT // _C):
        pltpu.sync_copy(rows_v, acc_sh.at[pl.ds(zbase + r * _C, _C)])
    _rem = _RPT - (_RPT // _C) * _C
    pltpu.sync_copy(rows_v.at[pl.ds(0, _rem)],
                    acc_sh.at[pl.ds(zbase + (_RPT // _C) * _C, _rem)])
    plsc.subcore_barrier()

    def _chunk(k, carry):
        base = wid * _EPW + k * _C
        pltpu.sync_copy(src_hbm.at[pl.ds(base, _C)], src_v)
        pltpu.sync_copy(dst_hbm.at[pl.ds(base, _C)], dst_v)
        for i in range(_C // 16):
            sl = pl.ds(i * 16, 16)
            widx_v[sl] = src_v[sl] * _N + dst_v[sl]
        d0 = pltpu.async_copy(wflat_hbm.at[widx_v], wv_v, sem0)
        d1 = pltpu.async_copy(t_hbm.at[src_v], rows_v, sem1)
        d0.wait()
        d1.wait()

        def _scale(e, cc):
            wsc = wv_v[e]
            for j in range(_D // 16):
                sl = pl.ds(j * 16, 16)
                rows_v[e, sl] = rows_v[e, sl] * wsc
            return cc

        lax.fori_loop(0, _C, _scale, 0)
        pltpu.sync_copy(rows_v, acc_sh.at[dst_v], add=True)
        return carry

    lax.fori_loop(0, _NCHUNK, _chunk, 0)
    plsc.subcore_barrier()

    pltpu.sync_copy(acc_sh.at[pl.ds(zbase, _RPT)],
                    out_hbm.at[c, pl.ds(zbase, _RPT)])


def kernel(values, edge_index, w):
    src = edge_index[0]
    dst = edge_index[1]
    wflat = w.reshape(_N * _N)

    t = pl.pallas_call(
        _tanh_body,
        grid=(_N // _TC_BR,),
        in_specs=[pl.BlockSpec((_TC_BR, _D), lambda i: (i, 0))],
        out_specs=pl.BlockSpec((_TC_BR, _D), lambda i: (i, 0)),
        out_shape=jax.ShapeDtypeStruct((_N, _D), jnp.float32),
    )(values)

    partials = _sc_scatter(t, src, dst, wflat)

    out = pl.pallas_call(
        _add_body,
        grid=(_N // _TC_BR,),
        in_specs=[
            pl.BlockSpec((1, _TC_BR, _D), lambda i: (0, i, 0)),
            pl.BlockSpec((1, _TC_BR, _D), lambda i: (1, i, 0)),
        ],
        out_specs=pl.BlockSpec((_TC_BR, _D), lambda i: (i, 0)),
        out_shape=jax.ShapeDtypeStruct((_N, _D), jnp.float32),
    )(partials, partials)
    return out


# speedup vs baseline: 2.2880x; 2.2880x over previous
import functools

import jax
import jax.numpy as jnp
from jax import lax
from jax.experimental import pallas as pl
from jax.experimental.pallas import tpu as pltpu
from jax.experimental.pallas import tpu_sc as plsc

_N = 10000
_E = 320000
_D = 128
_NC = 2
_NS = 16
_NW = _NC * _NS
_EPW = _E // _NW
_C = 80
_NCHUNK = _EPW // _C
_RPT = _N // _NS
_WBR = 624
_TC_BR = 1000


def _tanh_body(x_ref, o_ref):
    o_ref[...] = jnp.tanh(x_ref[...])


def _add_body(a_ref, b_ref, o_ref):
    o_ref[...] = a_ref[0] + b_ref[0]


_sc_mesh = plsc.VectorSubcoreMesh(core_axis_name="c", subcore_axis_name="s")


@functools.partial(
    pl.kernel,
    out_type=jax.ShapeDtypeStruct((_NC, _N, _D), jnp.float32),
    mesh=_sc_mesh,
    compiler_params=pltpu.CompilerParams(needs_layout_passes=False),
    scratch_types=[
        pltpu.VMEM((_C,), jnp.int32),
        pltpu.VMEM((_C,), jnp.int32),
        pltpu.VMEM((_C,), jnp.int32),
        pltpu.VMEM((_C,), jnp.float32),
        pltpu.VMEM((_C, _D), jnp.float32),
        pltpu.VMEM_SHARED((_N, _D), jnp.float32),
        pltpu.SemaphoreType.DMA,
        pltpu.SemaphoreType.DMA,
    ],
)
def _sc_scatter(t_hbm, src_hbm, dst_hbm, wflat_hbm, out_hbm,
                src_v, dst_v, widx_v, wv_v, rows_v, acc_sh, sem0, sem1):
    c = lax.axis_index("c")
    s = lax.axis_index("s")
    wid = c * _NS + s

    def _zrow(e, carry):
        z = jnp.zeros((16,), jnp.float32)
        for j in range(_D // 16):
            rows_v[e, pl.ds(j * 16, 16)] = z
        return carry

    lax.fori_loop(0, _C, _zrow, 0)
    zbase = s * _RPT
    for r in range(_RPT // _C):
        pltpu.sync_copy(rows_v, acc_sh.at[pl.ds(zbase + r * _C, _C)])
    _rem = _RPT - (_RPT // _C) * _C
    pltpu.sync_copy(rows_v.at[pl.ds(0, _rem)],
                    acc_sh.at[pl.ds(zbase + (_RPT // _C) * _C, _rem)])
    plsc.subcore_barrier()

    def _chunk(k, carry):
        base = wid * _EPW + k * _C
        pltpu.sync_copy(src_hbm.at[pl.ds(base, _C)], src_v)
        pltpu.sync_copy(dst_hbm.at[pl.ds(base, _C)], dst_v)
        for i in range(_C // 16):
            sl = pl.ds(i * 16, 16)
            widx_v[sl] = src_v[sl] * _N + dst_v[sl]
        d0 = pltpu.async_copy(wflat_hbm.at[widx_v], wv_v, sem0)
        d1 = pltpu.async_copy(t_hbm.at[src_v], rows_v, sem1)
        d0.wait()
        d1.wait()

        def _scale(e, cc):
            eidx = jnp.full((16,), e, jnp.int32)
            wsc = plsc.load_gather(wv_v, [eidx])
            for j in range(_D // 16):
                sl = pl.ds(j * 16, 16)
                rows_v[e, sl] = rows_v[e, sl] * wsc
            return cc

        lax.fori_loop(0, _C, _scale, 0)
        pltpu.sync_copy(rows_v, acc_sh.at[dst_v], add=True)
        return carry

    lax.fori_loop(0, _NCHUNK, _chunk, 0)
    plsc.subcore_barrier()

    wb = s * _WBR
    pltpu.sync_copy(acc_sh.at[pl.ds(wb, _WBR)],
                    out_hbm.at[c, pl.ds(wb, _WBR)])

    @pl.when(s == _NS - 1)
    def _tail():
        pltpu.sync_copy(acc_sh.at[pl.ds(_NS * _WBR, _N - _NS * _WBR)],
                        out_hbm.at[c, pl.ds(_NS * _WBR, _N - _NS * _WBR)])


def kernel(values, edge_index, w):
    src = edge_index[0]
    dst = edge_index[1]
    wflat = w.reshape(_N * _N)

    t = pl.pallas_call(
        _tanh_body,
        grid=(_N // _TC_BR,),
        in_specs=[pl.BlockSpec((_TC_BR, _D), lambda i: (i, 0))],
        out_specs=pl.BlockSpec((_TC_BR, _D), lambda i: (i, 0)),
        out_shape=jax.ShapeDtypeStruct((_N, _D), jnp.float32),
    )(values)

    partials = _sc_scatter(t, src, dst, wflat)

    out = pl.pallas_call(
        _add_body,
        grid=(_N // _TC_BR,),
        in_specs=[
            pl.BlockSpec((1, _TC_BR, _D), lambda i: (0, i, 0)),
            pl.BlockSpec((1, _TC_BR, _D), lambda i: (1, i, 0)),
        ],
        out_specs=pl.BlockSpec((_TC_BR, _D), lambda i: (i, 0)),
        out_shape=jax.ShapeDtypeStruct((_N, _D), jnp.float32),
    )(partials, partials)
    return out


# speedup vs baseline: 3.1973x; 1.3974x over previous
import functools

import jax
import jax.numpy as jnp
from jax import lax
from jax.experimental import pallas as pl
from jax.experimental.pallas import tpu as pltpu
from jax.experimental.pallas import tpu_sc as plsc

_N = 10000
_E = 320000
_D = 128
_NC = 2
_NS = 16
_NW = _NC * _NS
_EPW = _E // _NW
_C = 80
_NCHUNK = _EPW // _C
_RPT = _N // _NS
_WBR = 624
_TC_BR = 1000


def _tanh_body(x_ref, o_ref):
    o_ref[...] = jnp.tanh(x_ref[...])


def _add_body(a_ref, b_ref, o_ref):
    o_ref[...] = a_ref[0] + b_ref[0]


_sc_mesh = plsc.VectorSubcoreMesh(core_axis_name="c", subcore_axis_name="s")


@functools.partial(
    pl.kernel,
    out_type=jax.ShapeDtypeStruct((_NC, _N, _D), jnp.float32),
    mesh=_sc_mesh,
    compiler_params=pltpu.CompilerParams(needs_layout_passes=False),
    scratch_types=[
        pltpu.VMEM((_EPW,), jnp.int32),
        pltpu.VMEM((_EPW,), jnp.int32),
        pltpu.VMEM((_EPW,), jnp.int32),
        [pltpu.VMEM((_C,), jnp.float32)] * 2,
        [pltpu.VMEM((_C, _D), jnp.float32)] * 2,
        pltpu.VMEM_SHARED((_N, _D), jnp.float32),
        [pltpu.SemaphoreType.DMA] * 2,
        [pltpu.SemaphoreType.DMA] * 2,
        [pltpu.SemaphoreType.DMA] * 2,
    ],
)
def _sc_scatter(t_hbm, src_hbm, dst_hbm, wflat_hbm, out_hbm,
                src_v, dst_v, widx_v, wvs, rowss, acc_sh, gw, gt, sc):
    c = lax.axis_index("c")
    s = lax.axis_index("s")
    wid = c * _NS + s

    pltpu.sync_copy(src_hbm.at[pl.ds(wid * _EPW, _EPW)], src_v)
    pltpu.sync_copy(dst_hbm.at[pl.ds(wid * _EPW, _EPW)], dst_v)

    def _widx_row(i, carry):
        sl = pl.ds(i * 16, 16)
        widx_v[sl] = src_v[sl] * _N + dst_v[sl]
        return carry

    lax.fori_loop(0, _EPW // 16, _widx_row, 0)

    def _zrow(e, carry):
        z = jnp.zeros((16,), jnp.float32)
        for j in range(_D // 16):
            rowss[0][e, pl.ds(j * 16, 16)] = z
        return carry

    lax.fori_loop(0, _C, _zrow, 0)
    zbase = s * _RPT
    for r in range(_RPT // _C):
        pltpu.sync_copy(rowss[0], acc_sh.at[pl.ds(zbase + r * _C, _C)])
    _rem = _RPT - (_RPT // _C) * _C
    pltpu.sync_copy(rowss[0].at[pl.ds(0, _rem)],
                    acc_sh.at[pl.ds(zbase + (_RPT // _C) * _C, _rem)])
    plsc.subcore_barrier()

    def _issue(k, b):
        pltpu.async_copy(wflat_hbm.at[widx_v.at[pl.ds(k * _C, _C)]], wvs[b], gw[b])
        pltpu.async_copy(t_hbm.at[src_v.at[pl.ds(k * _C, _C)]], rowss[b], gt[b])

    def _wait_gathers(k, b):
        pltpu.make_async_copy(wflat_hbm.at[widx_v.at[pl.ds(k * _C, _C)]], wvs[b], gw[b]).wait()
        pltpu.make_async_copy(t_hbm.at[src_v.at[pl.ds(k * _C, _C)]], rowss[b], gt[b]).wait()

    def _scale(b):
        def _srow(e, cc):
            eidx = jnp.full((16,), e, jnp.int32)
            wsc = plsc.load_gather(wvs[b], [eidx])
            for j in range(_D // 16):
                sl = pl.ds(j * 16, 16)
                rowss[b][e, sl] = rowss[b][e, sl] * wsc
            return cc

        lax.fori_loop(0, _C, _srow, 0)

    def _scatter(k, b):
        pltpu.async_copy(rowss[b], acc_sh.at[dst_v.at[pl.ds(k * _C, _C)]], sc[b], add=True)

    def _wait_scatter(k, b):
        pltpu.make_async_copy(rowss[b], acc_sh.at[dst_v.at[pl.ds(k * _C, _C)]], sc[b]).wait()

    def _steady(k, b):
        bn = b ^ 1
        _wait_scatter(k - 1, bn)
        _issue(k + 1, bn)
        _wait_gathers(k, b)
        _scale(b)
        _scatter(k, b)

    _issue(0, 0)
    _issue(1, 1)
    _wait_gathers(0, 0)
    _scale(0)
    _scatter(0, 0)

    def _pair(i, carry):
        k = 2 * i + 1
        _steady(k, 1)
        _steady(k + 1, 0)
        return carry

    lax.fori_loop(0, (_NCHUNK - 3) // 2, _pair, 0)
    _steady(_NCHUNK - 2, 1)
    _wait_scatter(_NCHUNK - 2, 1)
    _wait_gathers(_NCHUNK - 1, 0)
    _scale(0)
    _scatter(_NCHUNK - 1, 0)
    _wait_scatter(_NCHUNK - 1, 0)

    plsc.subcore_barrier()

    wb = s * _WBR
    pltpu.sync_copy(acc_sh.at[pl.ds(wb, _WBR)],
                    out_hbm.at[c, pl.ds(wb, _WBR)])

    @pl.when(s == _NS - 1)
    def _tail():
        pltpu.sync_copy(acc_sh.at[pl.ds(_NS * _WBR, _N - _NS * _WBR)],
                        out_hbm.at[c, pl.ds(_NS * _WBR, _N - _NS * _WBR)])


def kernel(values, edge_index, w):
    src = edge_index[0]
    dst = edge_index[1]
    wflat = w.reshape(_N * _N)

    t = pl.pallas_call(
        _tanh_body,
        grid=(_N // _TC_BR,),
        in_specs=[pl.BlockSpec((_TC_BR, _D), lambda i: (i, 0))],
        out_specs=pl.BlockSpec((_TC_BR, _D), lambda i: (i, 0)),
        out_shape=jax.ShapeDtypeStruct((_N, _D), jnp.float32),
    )(values)

    partials = _sc_scatter(t, src, dst, wflat)

    out = pl.pallas_call(
        _add_body,
        grid=(_N // _TC_BR,),
        in_specs=[
            pl.BlockSpec((1, _TC_BR, _D), lambda i: (0, i, 0)),
            pl.BlockSpec((1, _TC_BR, _D), lambda i: (1, i, 0)),
        ],
        out_specs=pl.BlockSpec((_TC_BR, _D), lambda i: (i, 0)),
        out_shape=jax.ShapeDtypeStruct((_N, _D), jnp.float32),
    )(partials, partials)
    return out
